# baseline (speedup 1.0000x reference)
import functools

import jax
import jax.numpy as jnp
from jax import lax
from jax.experimental import pallas as pl
from jax.experimental.pallas import tpu as pltpu
from jax.experimental.pallas import tpu_sc as plsc

_N_VN = 10000
_N_CN = 5000
_E = 30000
_B = 1250
_NUM_ITER = 5

_BP = 1280
_NW = 32
_ROWS_PER_TILE = 960
_EP = _NW * _ROWS_PER_TILE
_VNP = 10240
_GCHUNK = 48
_NGCHUNK = _ROWS_PER_TILE // _GCHUNK

_ABLK = 480
_VBLK = 480


def _make_gather(table_rows):
    del table_rows
    mesh = plsc.VectorSubcoreMesh(core_axis_name="c", subcore_axis_name="s")

    @functools.partial(
        pl.kernel,
        mesh=mesh,
        out_type=jax.ShapeDtypeStruct((_EP, _BP), jnp.float32),
        scratch_types=[
            pltpu.VMEM((_GCHUNK,), jnp.int32),
            pltpu.VMEM((_GCHUNK, _BP), jnp.float32),
            pltpu.SemaphoreType.DMA,
        ],
        compiler_params=pltpu.CompilerParams(
            needs_layout_passes=False, use_tc_tiling_on_sc=True),
    )
    def gather(table, idx, out, idx_v, buf, sem):
        wid = lax.axis_index("s") * 2 + lax.axis_index("c")
        tile_base = wid * _ROWS_PER_TILE

        def chunk_body(ci, carry):
            rbase = tile_base + ci * _GCHUNK
            pltpu.sync_copy(idx.at[pl.ds(rbase, _GCHUNK)], idx_v)
            pltpu.async_copy(table.at[idx_v], buf, sem).wait()
            pltpu.sync_copy(buf, out.at[pl.ds(rbase, _GCHUNK)])
            return carry

        lax.fori_loop(0, _NGCHUNK, chunk_body, 0)

    return gather


def _boxplus_tc_kernel(g_ref, w_ref, o_ref):
    x = g_ref[...]
    w = w_ref[...]
    t = jnp.tanh(jnp.clip(x, -20.0, 20.0) * 0.5)
    logmag = jnp.log(jnp.abs(t) + 1e-12)
    neg = (t < 0.0).astype(jnp.int32)
    lm3 = logmag.reshape(_ABLK // 6, 6, _BP)
    ext_lm = (jnp.sum(lm3, axis=1, keepdims=True) - lm3).reshape(_ABLK, _BP)
    n3 = neg.reshape(_ABLK // 6, 6, _BP)
    extn = (jnp.sum(n3, axis=1, keepdims=True) - n3).reshape(_ABLK, _BP)
    sign = 1.0 - 2.0 * (extn % 2).astype(jnp.float32)
    y = jnp.clip(sign * jnp.exp(ext_lm), -1.0 + 1e-7, 1.0 - 1e-7)
    o_ref[...] = w * jnp.log((1.0 + y) / (1.0 - y))


def _boxplus_tc(g, w2d):
    return pl.pallas_call(
        _boxplus_tc_kernel,
        out_shape=jax.ShapeDtypeStruct((_EP, _BP), jnp.float32),
        grid=(_EP // _ABLK,),
        in_specs=[
            pl.BlockSpec((_ABLK, _BP), lambda i: (i, 0)),
            pl.BlockSpec((_ABLK, 1), lambda i: (i, 0)),
        ],
        out_specs=pl.BlockSpec((_ABLK, _BP), lambda i: (i, 0)),
    )(g, w2d)


def _vnupd_tc_kernel(h_ref, l_ref, msg_ref, marg_ref, lp_ref):
    i = pl.program_id(0)
    vrows = _VBLK // 3
    h = h_ref[...]
    l = l_ref[...]
    h3 = h.reshape(vrows, 3, _BP)
    marg = l + jnp.sum(h3, axis=1)
    marg_ref[...] = marg
    msg_ref[...] = (marg.reshape(vrows, 1, _BP) - h3).reshape(_VBLK, _BP)
    rows = lax.broadcasted_iota(jnp.int32, (vrows, _BP), 0) + i * vrows
    cols = lax.broadcasted_iota(jnp.int32, (vrows, _BP), 1)
    mask = (rows < _N_VN) & (cols < _B)
    sp = jnp.where(mask, jax.nn.softplus(marg), 0.0)

    @pl.when(i == 0)
    def _():
        lp_ref[...] = jnp.zeros_like(lp_ref)

    lp_ref[...] += jnp.sum(sp, axis=0, keepdims=True)


def _vnupd_tc(h, llr_t):
    vrows = _VBLK // 3
    return pl.pallas_call(
        _vnupd_tc_kernel,
        out_shape=(
            jax.ShapeDtypeStruct((_EP, _BP), jnp.float32),
            jax.ShapeDtypeStruct((_VNP, _BP), jnp.float32),
            jax.ShapeDtypeStruct((1, _BP), jnp.float32),
        ),
        grid=(_EP // _VBLK,),
        in_specs=[
            pl.BlockSpec((_VBLK, _BP), lambda i: (i, 0)),
            pl.BlockSpec((vrows, _BP), lambda i: (i, 0)),
        ],
        out_specs=(
            pl.BlockSpec((_VBLK, _BP), lambda i: (i, 0)),
            pl.BlockSpec((vrows, _BP), lambda i: (i, 0)),
            pl.BlockSpec((1, _BP), lambda i: (0, 0)),
        ),
    )(h, llr_t)


def kernel(llr, weights, vn_idx, cn_idx):
    llr = llr.astype(jnp.float32)
    b = llr.shape[0]

    perm = jnp.argsort(cn_idx).astype(jnp.int32)
    inv = jnp.zeros((_E,), jnp.int32).at[perm].set(jnp.arange(_E, dtype=jnp.int32))
    zpad = jnp.zeros((_EP - _E,), jnp.int32)
    perm3_pad = jnp.concatenate([vn_idx[perm].astype(jnp.int32), zpad])
    perm_pad = jnp.concatenate([perm, zpad])
    inv_pad = jnp.concatenate([inv, zpad])
    w2d = jnp.concatenate([weights[perm].astype(jnp.float32),
                           jnp.ones((_EP - _E,), jnp.float32)])[:, None]

    llr_t = jnp.pad(llr.T, ((0, _VNP - _N_VN), (0, _BP - b)))

    gather_first = _make_gather(_VNP)
    gather_rest = _make_gather(_EP)
    gather_back = _make_gather(_EP)

    loss_parts = []
    marg = None
    msg_vn = None
    for it in range(_NUM_ITER):
        if it == 0:
            g = gather_first(llr_t, perm3_pad)
        else:
            g = gather_rest(msg_vn, perm_pad)
        msg_cn = _boxplus_tc(g, w2d)
        h = gather_back(msg_cn, inv_pad)
        msg_vn, marg, lp = _vnupd_tc(h, llr_t)
        loss_parts.append(lp)

    loss = jnp.sum(jnp.stack(loss_parts)) / jnp.float32(_NUM_ITER * b * _N_VN)
    c_hat = marg[:_N_VN, :b].T
    c = jnp.zeros((b, _N_VN), dtype=jnp.float32)
    return (c, c_hat, llr, loss)


# speedup vs baseline: 1.1354x; 1.1354x over previous
import functools

import jax
import jax.numpy as jnp
from jax import lax
from jax.experimental import pallas as pl
from jax.experimental.pallas import tpu as pltpu
from jax.experimental.pallas import tpu_sc as plsc

_N_VN = 10000
_N_CN = 5000
_E = 30000
_B = 1250
_NUM_ITER = 5

_BP = 1280
_NW = 32
_ROWS_PER_TILE = 960
_EP = _NW * _ROWS_PER_TILE
_VNP = 10240
_GCHUNK = 48
_NGCHUNK = _ROWS_PER_TILE // _GCHUNK

_ABLK = 480
_VBLK = 480


def _make_gather(table_rows):
    del table_rows
    mesh = plsc.VectorSubcoreMesh(core_axis_name="c", subcore_axis_name="s")

    @functools.partial(
        pl.kernel,
        mesh=mesh,
        out_type=jax.ShapeDtypeStruct((_EP, _BP), jnp.float32),
        scratch_types=[
            pltpu.VMEM((_GCHUNK,), jnp.int32),
            pltpu.VMEM((_GCHUNK, _BP), jnp.float32),
            pltpu.SemaphoreType.DMA,
        ],
        compiler_params=pltpu.CompilerParams(
            needs_layout_passes=False, use_tc_tiling_on_sc=True),
    )
    def gather(table, idx, out, idx_v, buf, sem):
        wid = lax.axis_index("s") * 2 + lax.axis_index("c")
        tile_base = wid * _ROWS_PER_TILE

        def chunk_body(ci, carry):
            rbase = tile_base + ci * _GCHUNK
            pltpu.sync_copy(idx.at[pl.ds(rbase, _GCHUNK)], idx_v)
            pltpu.async_copy(table.at[idx_v], buf, sem).wait()
            pltpu.sync_copy(buf, out.at[pl.ds(rbase, _GCHUNK)])
            return carry

        lax.fori_loop(0, _NGCHUNK, chunk_body, 0)

    return gather


def _boxplus_tc_kernel(g_ref, w_ref, o_ref):
    x = g_ref[...]
    w = w_ref[...]
    t = jnp.tanh(jnp.clip(x, -20.0, 20.0) * 0.5)
    u = t + jnp.where(t < 0.0, jnp.float32(-1e-12), jnp.float32(1e-12))
    u3 = u.reshape(_ABLK // 6, 6, _BP)
    us = [u3[:, k, :] for k in range(6)]
    pre = [us[0]]
    for k in range(1, 5):
        pre.append(pre[-1] * us[k])
    suf = [None] * 6
    suf[5] = us[5]
    for k in range(4, 0, -1):
        suf[k] = suf[k + 1] * us[k]
    exts = []
    for k in range(6):
        if k == 0:
            exts.append(suf[1])
        elif k == 5:
            exts.append(pre[4])
        else:
            exts.append(pre[k - 1] * suf[k + 1])
    ext = jnp.stack(exts, axis=1).reshape(_ABLK, _BP)
    y = jnp.clip(ext, -1.0 + 1e-7, 1.0 - 1e-7)
    o_ref[...] = w * jnp.log((1.0 + y) / (1.0 - y))


def _boxplus_tc(g, w2d):
    return pl.pallas_call(
        _boxplus_tc_kernel,
        out_shape=jax.ShapeDtypeStruct((_EP, _BP), jnp.float32),
        grid=(_EP // _ABLK,),
        in_specs=[
            pl.BlockSpec((_ABLK, _BP), lambda i: (i, 0)),
            pl.BlockSpec((_ABLK, 1), lambda i: (i, 0)),
        ],
        out_specs=pl.BlockSpec((_ABLK, _BP), lambda i: (i, 0)),
    )(g, w2d)


def _vnupd_tc_kernel(h_ref, l_ref, msg_ref, marg_ref, lp_ref):
    i = pl.program_id(0)
    vrows = _VBLK // 3
    h = h_ref[...]
    l = l_ref[...]
    h3 = h.reshape(vrows, 3, _BP)
    marg = l + jnp.sum(h3, axis=1)
    marg_ref[...] = marg
    msg_ref[...] = (marg.reshape(vrows, 1, _BP) - h3).reshape(_VBLK, _BP)
    rows = lax.broadcasted_iota(jnp.int32, (vrows, _BP), 0) + i * vrows
    cols = lax.broadcasted_iota(jnp.int32, (vrows, _BP), 1)
    mask = (rows < _N_VN) & (cols < _B)
    sp = jnp.where(mask, jax.nn.softplus(marg), 0.0)

    @pl.when(i == 0)
    def _():
        lp_ref[...] = jnp.zeros_like(lp_ref)

    lp_ref[...] += jnp.sum(sp, axis=0, keepdims=True)


def _vnupd_tc(h, llr_t):
    vrows = _VBLK // 3
    return pl.pallas_call(
        _vnupd_tc_kernel,
        out_shape=(
            jax.ShapeDtypeStruct((_EP, _BP), jnp.float32),
            jax.ShapeDtypeStruct((_VNP, _BP), jnp.float32),
            jax.ShapeDtypeStruct((1, _BP), jnp.float32),
        ),
        grid=(_EP // _VBLK,),
        in_specs=[
            pl.BlockSpec((_VBLK, _BP), lambda i: (i, 0)),
            pl.BlockSpec((vrows, _BP), lambda i: (i, 0)),
        ],
        out_specs=(
            pl.BlockSpec((_VBLK, _BP), lambda i: (i, 0)),
            pl.BlockSpec((vrows, _BP), lambda i: (i, 0)),
            pl.BlockSpec((1, _BP), lambda i: (0, 0)),
        ),
    )(h, llr_t)


def kernel(llr, weights, vn_idx, cn_idx):
    llr = llr.astype(jnp.float32)
    b = llr.shape[0]

    perm = jnp.argsort(cn_idx).astype(jnp.int32)
    inv = jnp.zeros((_E,), jnp.int32).at[perm].set(jnp.arange(_E, dtype=jnp.int32))
    zpad = jnp.zeros((_EP - _E,), jnp.int32)
    perm3_pad = jnp.concatenate([vn_idx[perm].astype(jnp.int32), zpad])
    perm_pad = jnp.concatenate([perm, zpad])
    inv_pad = jnp.concatenate([inv, zpad])
    w2d = jnp.concatenate([weights[perm].astype(jnp.float32),
                           jnp.ones((_EP - _E,), jnp.float32)])[:, None]

    llr_t = jnp.pad(llr.T, ((0, _VNP - _N_VN), (0, _BP - b)))

    gather_first = _make_gather(_VNP)
    gather_rest = _make_gather(_EP)
    gather_back = _make_gather(_EP)

    loss_parts = []
    marg = None
    msg_vn = None
    for it in range(_NUM_ITER):
        if it == 0:
            g = gather_first(llr_t, perm3_pad)
        else:
            g = gather_rest(msg_vn, perm_pad)
        msg_cn = _boxplus_tc(g, w2d)
        h = gather_back(msg_cn, inv_pad)
        msg_vn, marg, lp = _vnupd_tc(h, llr_t)
        loss_parts.append(lp)

    loss = jnp.sum(jnp.stack(loss_parts)) / jnp.float32(_NUM_ITER * b * _N_VN)
    c_hat = marg[:_N_VN, :b].T
    c = jnp.zeros((b, _N_VN), dtype=jnp.float32)
    return (c, c_hat, llr, loss)


# speedup vs baseline: 1.2409x; 1.0929x over previous
import functools

import jax
import jax.numpy as jnp
from jax import lax
from jax.experimental import pallas as pl
from jax.experimental.pallas import tpu as pltpu
from jax.experimental.pallas import tpu_sc as plsc

_N_VN = 10000
_N_CN = 5000
_E = 30000
_B = 1250
_NUM_ITER = 5

_BP = 1280
_NW = 32
_ROWS_PER_TILE = 960
_EP = _NW * _ROWS_PER_TILE
_VNP = 10240
_GCHUNK = 48
_NGCHUNK = _ROWS_PER_TILE // _GCHUNK

_ABLK = 480
_VBLK = 480


def _make_gather(table_rows):
    del table_rows
    mesh = plsc.VectorSubcoreMesh(core_axis_name="c", subcore_axis_name="s")

    @functools.partial(
        pl.kernel,
        mesh=mesh,
        out_type=jax.ShapeDtypeStruct((_EP, _BP), jnp.float32),
        scratch_types=[
            pltpu.VMEM((_GCHUNK,), jnp.int32),
            pltpu.VMEM((_GCHUNK, _BP), jnp.float32),
            pltpu.SemaphoreType.DMA,
        ],
        compiler_params=pltpu.CompilerParams(
            needs_layout_passes=False, use_tc_tiling_on_sc=True),
    )
    def gather(table, idx, out, idx_v, buf, sem):
        wid = lax.axis_index("s") * 2 + lax.axis_index("c")
        tile_base = wid * _ROWS_PER_TILE

        def chunk_body(ci, carry):
            rbase = tile_base + ci * _GCHUNK
            pltpu.sync_copy(idx.at[pl.ds(rbase, _GCHUNK)], idx_v)
            pltpu.async_copy(table.at[idx_v], buf, sem).wait()
            pltpu.sync_copy(buf, out.at[pl.ds(rbase, _GCHUNK)])
            return carry

        lax.fori_loop(0, _NGCHUNK, chunk_body, 0)

    return gather


def _boxplus_tc_kernel(g_ref, w_ref, o_ref):
    _boxplus_body(g_ref[...], w_ref[...], o_ref)


def _boxplus_sub_tc_kernel(g_ref, p_ref, w_ref, o_ref):
    _boxplus_body(g_ref[...] - p_ref[...], w_ref[...], o_ref)


def _boxplus_body(x, w, o_ref):
    t = jnp.tanh(jnp.clip(x, -20.0, 20.0) * 0.5)
    u = t + jnp.where(t < 0.0, jnp.float32(-1e-12), jnp.float32(1e-12))
    u3 = u.reshape(_ABLK // 6, 6, _BP)
    us = [u3[:, k, :] for k in range(6)]
    pre = [us[0]]
    for k in range(1, 5):
        pre.append(pre[-1] * us[k])
    suf = [None] * 6
    suf[5] = us[5]
    for k in range(4, 0, -1):
        suf[k] = suf[k + 1] * us[k]
    exts = []
    for k in range(6):
        if k == 0:
            exts.append(suf[1])
        elif k == 5:
            exts.append(pre[4])
        else:
            exts.append(pre[k - 1] * suf[k + 1])
    ext = jnp.stack(exts, axis=1).reshape(_ABLK, _BP)
    y = jnp.clip(ext, -1.0 + 1e-7, 1.0 - 1e-7)
    o_ref[...] = w * jnp.log((1.0 + y) / (1.0 - y))


def _boxplus_tc(g, w2d, prev=None):
    blk = pl.BlockSpec((_ABLK, _BP), lambda i: (i, 0))
    wblk = pl.BlockSpec((_ABLK, 1), lambda i: (i, 0))
    if prev is None:
        return pl.pallas_call(
            _boxplus_tc_kernel,
            out_shape=jax.ShapeDtypeStruct((_EP, _BP), jnp.float32),
            grid=(_EP // _ABLK,),
            in_specs=[blk, wblk],
            out_specs=blk,
        )(g, w2d)
    return pl.pallas_call(
        _boxplus_sub_tc_kernel,
        out_shape=jax.ShapeDtypeStruct((_EP, _BP), jnp.float32),
        grid=(_EP // _ABLK,),
        in_specs=[blk, blk, wblk],
        out_specs=blk,
    )(g, prev, w2d)


def _vnupd_tc_kernel(h_ref, l_ref, marg_ref, lp_ref):
    i = pl.program_id(0)
    vrows = _VBLK // 3
    h = h_ref[...]
    l = l_ref[...]
    h3 = h.reshape(vrows, 3, _BP)
    marg = l + jnp.sum(h3, axis=1)
    marg_ref[...] = marg
    rows = lax.broadcasted_iota(jnp.int32, (vrows, _BP), 0) + i * vrows
    cols = lax.broadcasted_iota(jnp.int32, (vrows, _BP), 1)
    mask = (rows < _N_VN) & (cols < _B)
    sp = jnp.where(mask, jax.nn.softplus(marg), 0.0)

    @pl.when(i == 0)
    def _():
        lp_ref[...] = jnp.zeros_like(lp_ref)

    lp_ref[...] += jnp.sum(sp, axis=0, keepdims=True)


def _vnupd_tc(h, llr_t):
    vrows = _VBLK // 3
    return pl.pallas_call(
        _vnupd_tc_kernel,
        out_shape=(
            jax.ShapeDtypeStruct((_VNP, _BP), jnp.float32),
            jax.ShapeDtypeStruct((1, _BP), jnp.float32),
        ),
        grid=(_EP // _VBLK,),
        in_specs=[
            pl.BlockSpec((_VBLK, _BP), lambda i: (i, 0)),
            pl.BlockSpec((vrows, _BP), lambda i: (i, 0)),
        ],
        out_specs=(
            pl.BlockSpec((vrows, _BP), lambda i: (i, 0)),
            pl.BlockSpec((1, _BP), lambda i: (0, 0)),
        ),
    )(h, llr_t)


def kernel(llr, weights, vn_idx, cn_idx):
    llr = llr.astype(jnp.float32)
    b = llr.shape[0]

    perm = jnp.argsort(cn_idx).astype(jnp.int32)
    inv = jnp.zeros((_E,), jnp.int32).at[perm].set(jnp.arange(_E, dtype=jnp.int32))
    zpad = jnp.zeros((_EP - _E,), jnp.int32)
    perm3_pad = jnp.concatenate([vn_idx[perm].astype(jnp.int32), zpad])
    perm_pad = jnp.concatenate([perm, zpad])
    inv_pad = jnp.concatenate([inv, zpad])
    w2d = jnp.concatenate([weights[perm].astype(jnp.float32),
                           jnp.ones((_EP - _E,), jnp.float32)])[:, None]

    llr_t = jnp.pad(llr.T, ((0, _VNP - _N_VN), (0, _BP - b)))

    gather_marg = _make_gather(_VNP)
    gather_back = _make_gather(_EP)

    loss_parts = []
    marg = None
    msg_cn = None
    for it in range(_NUM_ITER):
        if it == 0:
            g = gather_marg(llr_t, perm3_pad)
            msg_cn = _boxplus_tc(g, w2d)
        else:
            g = gather_marg(marg, perm3_pad)
            msg_cn = _boxplus_tc(g, w2d, prev=msg_cn)
        h = gather_back(msg_cn, inv_pad)
        marg, lp = _vnupd_tc(h, llr_t)
        loss_parts.append(lp)

    loss = jnp.sum(jnp.stack(loss_parts)) / jnp.float32(_NUM_ITER * b * _N_VN)
    c_hat = marg[:_N_VN, :b].T
    c = jnp.zeros((b, _N_VN), dtype=jnp.float32)
    return (c, c_hat, llr, loss)


# speedup vs baseline: 1.2489x; 1.0065x over previous
import functools

import jax
import jax.numpy as jnp
from jax import lax
from jax.experimental import pallas as pl
from jax.experimental.pallas import tpu as pltpu
from jax.experimental.pallas import tpu_sc as plsc

_N_VN = 10000
_N_CN = 5000
_E = 30000
_B = 1250
_NUM_ITER = 5

_BP = 1280
_NW = 32
_ROWS_PER_TILE = 960
_EP = _NW * _ROWS_PER_TILE
_VNP = 10240
_GCHUNK = 80
_NGCHUNK = _ROWS_PER_TILE // _GCHUNK

_ABLK = 480
_VBLK = 480


def _make_gather(table_rows):
    del table_rows
    mesh = plsc.VectorSubcoreMesh(core_axis_name="c", subcore_axis_name="s")

    @functools.partial(
        pl.kernel,
        mesh=mesh,
        out_type=jax.ShapeDtypeStruct((_EP, _BP), jnp.float32),
        scratch_types=[
            pltpu.VMEM((_GCHUNK,), jnp.int32),
            pltpu.VMEM((_GCHUNK, _BP), jnp.float32),
            pltpu.SemaphoreType.DMA,
        ],
        compiler_params=pltpu.CompilerParams(
            needs_layout_passes=False, use_tc_tiling_on_sc=True),
    )
    def gather(table, idx, out, idx_v, buf, sem):
        wid = lax.axis_index("s") * 2 + lax.axis_index("c")
        tile_base = wid * _ROWS_PER_TILE

        def chunk_body(ci, carry):
            rbase = tile_base + ci * _GCHUNK
            pltpu.sync_copy(idx.at[pl.ds(rbase, _GCHUNK)], idx_v)
            pltpu.async_copy(table.at[idx_v], buf, sem).wait()
            pltpu.sync_copy(buf, out.at[pl.ds(rbase, _GCHUNK)])
            return carry

        lax.fori_loop(0, _NGCHUNK, chunk_body, 0)

    return gather


def _boxplus_tc_kernel(g_ref, w_ref, o_ref):
    _boxplus_body(g_ref[...], w_ref[...], o_ref)


def _boxplus_sub_tc_kernel(g_ref, p_ref, w_ref, o_ref):
    _boxplus_body(g_ref[...] - p_ref[...], w_ref[...], o_ref)


def _boxplus_body(x, w, o_ref):
    t = jnp.tanh(jnp.clip(x, -20.0, 20.0) * 0.5)
    u = t + jnp.where(t < 0.0, jnp.float32(-1e-12), jnp.float32(1e-12))
    u3 = u.reshape(_ABLK // 6, 6, _BP)
    us = [u3[:, k, :] for k in range(6)]
    pre = [us[0]]
    for k in range(1, 5):
        pre.append(pre[-1] * us[k])
    suf = [None] * 6
    suf[5] = us[5]
    for k in range(4, 0, -1):
        suf[k] = suf[k + 1] * us[k]
    exts = []
    for k in range(6):
        if k == 0:
            exts.append(suf[1])
        elif k == 5:
            exts.append(pre[4])
        else:
            exts.append(pre[k - 1] * suf[k + 1])
    ext = jnp.stack(exts, axis=1).reshape(_ABLK, _BP)
    y = jnp.clip(ext, -1.0 + 1e-7, 1.0 - 1e-7)
    o_ref[...] = w * jnp.log((1.0 + y) / (1.0 - y))


def _boxplus_tc(g, w2d, prev=None):
    blk = pl.BlockSpec((_ABLK, _BP), lambda i: (i, 0))
    wblk = pl.BlockSpec((_ABLK, 1), lambda i: (i, 0))
    if prev is None:
        return pl.pallas_call(
            _boxplus_tc_kernel,
            out_shape=jax.ShapeDtypeStruct((_EP, _BP), jnp.float32),
            grid=(_EP // _ABLK,),
            in_specs=[blk, wblk],
            out_specs=blk,
        )(g, w2d)
    return pl.pallas_call(
        _boxplus_sub_tc_kernel,
        out_shape=jax.ShapeDtypeStruct((_EP, _BP), jnp.float32),
        grid=(_EP // _ABLK,),
        in_specs=[blk, blk, wblk],
        out_specs=blk,
    )(g, prev, w2d)


def _vnupd_tc_kernel(h_ref, l_ref, marg_ref, lp_ref):
    i = pl.program_id(0)
    vrows = _VBLK // 3
    h = h_ref[...]
    l = l_ref[...]
    h3 = h.reshape(vrows, 3, _BP)
    marg = l + jnp.sum(h3, axis=1)
    marg_ref[...] = marg
    rows = lax.broadcasted_iota(jnp.int32, (vrows, _BP), 0) + i * vrows
    cols = lax.broadcasted_iota(jnp.int32, (vrows, _BP), 1)
    mask = (rows < _N_VN) & (cols < _B)
    sp = jnp.where(mask, jax.nn.softplus(marg), 0.0)

    @pl.when(i == 0)
    def _():
        lp_ref[...] = jnp.zeros_like(lp_ref)

    lp_ref[...] += jnp.sum(sp, axis=0, keepdims=True)


def _vnupd_tc(h, llr_t):
    vrows = _VBLK // 3
    return pl.pallas_call(
        _vnupd_tc_kernel,
        out_shape=(
            jax.ShapeDtypeStruct((_VNP, _BP), jnp.float32),
            jax.ShapeDtypeStruct((1, _BP), jnp.float32),
        ),
        grid=(_EP // _VBLK,),
        in_specs=[
            pl.BlockSpec((_VBLK, _BP), lambda i: (i, 0)),
            pl.BlockSpec((vrows, _BP), lambda i: (i, 0)),
        ],
        out_specs=(
            pl.BlockSpec((vrows, _BP), lambda i: (i, 0)),
            pl.BlockSpec((1, _BP), lambda i: (0, 0)),
        ),
    )(h, llr_t)


def kernel(llr, weights, vn_idx, cn_idx):
    llr = llr.astype(jnp.float32)
    b = llr.shape[0]

    perm = jnp.argsort(cn_idx).astype(jnp.int32)
    inv = jnp.zeros((_E,), jnp.int32).at[perm].set(jnp.arange(_E, dtype=jnp.int32))
    zpad = jnp.zeros((_EP - _E,), jnp.int32)
    perm3_pad = jnp.concatenate([vn_idx[perm].astype(jnp.int32), zpad])
    perm_pad = jnp.concatenate([perm, zpad])
    inv_pad = jnp.concatenate([inv, zpad])
    w2d = jnp.concatenate([weights[perm].astype(jnp.float32),
                           jnp.ones((_EP - _E,), jnp.float32)])[:, None]

    llr_t = jnp.pad(llr.T, ((0, _VNP - _N_VN), (0, _BP - b)))

    gather_marg = _make_gather(_VNP)
    gather_back = _make_gather(_EP)

    loss_parts = []
    marg = None
    msg_cn = None
    for it in range(_NUM_ITER):
        if it == 0:
            g = gather_marg(llr_t, perm3_pad)
            msg_cn = _boxplus_tc(g, w2d)
        else:
            g = gather_marg(marg, perm3_pad)
            msg_cn = _boxplus_tc(g, w2d, prev=msg_cn)
        h = gather_back(msg_cn, inv_pad)
        marg, lp = _vnupd_tc(h, llr_t)
        loss_parts.append(lp)

    loss = jnp.sum(jnp.stack(loss_parts)) / jnp.float32(_NUM_ITER * b * _N_VN)
    c_hat = marg[:_N_VN, :b].T
    c = jnp.zeros((b, _N_VN), dtype=jnp.float32)
    return (c, c_hat, llr, loss)


# speedup vs baseline: 1.2490x; 1.0000x over previous
import functools

import jax
import jax.numpy as jnp
from jax import lax
from jax.experimental import pallas as pl
from jax.experimental.pallas import tpu as pltpu
from jax.experimental.pallas import tpu_sc as plsc

_N_VN = 10000
_N_CN = 5000
_E = 30000
_B = 1250
_NUM_ITER = 5

_BP = 1280
_NW = 32
_ROWS_PER_TILE = 960
_EP = _NW * _ROWS_PER_TILE
_VNP = 10240
_GCHUNK = 80
_NGCHUNK = _ROWS_PER_TILE // _GCHUNK

_ABLK = 480
_VBLK = 480


def _make_gather(table_rows):
    del table_rows
    mesh = plsc.VectorSubcoreMesh(core_axis_name="c", subcore_axis_name="s")

    @functools.partial(
        pl.kernel,
        mesh=mesh,
        out_type=jax.ShapeDtypeStruct((_EP, _BP), jnp.float32),
        scratch_types=[
            pltpu.VMEM((_GCHUNK,), jnp.int32),
            pltpu.VMEM((_GCHUNK, _BP), jnp.float32),
            pltpu.SemaphoreType.DMA,
        ],
        compiler_params=pltpu.CompilerParams(
            needs_layout_passes=False, use_tc_tiling_on_sc=True),
    )
    def gather(table, idx, out, idx_v, buf, sem):
        wid = lax.axis_index("s") * 2 + lax.axis_index("c")
        tile_base = wid * _ROWS_PER_TILE

        def chunk_body(ci, carry):
            rbase = tile_base + ci * _GCHUNK
            pltpu.sync_copy(idx.at[pl.ds(rbase, _GCHUNK)], idx_v)
            pltpu.async_copy(table.at[idx_v], buf, sem).wait()
            pltpu.sync_copy(buf, out.at[pl.ds(rbase, _GCHUNK)])
            return carry

        lax.fori_loop(0, _NGCHUNK, chunk_body, 0)

    return gather


def _boxplus_tc_kernel(g_ref, w_ref, o_ref):
    _boxplus_body(g_ref[...], w_ref[...], o_ref)


def _boxplus_sub_tc_kernel(g_ref, p_ref, w_ref, o_ref):
    _boxplus_body(g_ref[...] - p_ref[...], w_ref[...], o_ref)


def _boxplus_body(x, w, o_ref):
    t = jnp.tanh(jnp.clip(x, -20.0, 20.0) * 0.5)
    u = t + jnp.where(t < 0.0, jnp.float32(-1e-12), jnp.float32(1e-12))
    u3 = u.reshape(_ABLK // 6, 6, _BP)
    us = [u3[:, k, :] for k in range(6)]
    pre = [us[0]]
    for k in range(1, 5):
        pre.append(pre[-1] * us[k])
    suf = [None] * 6
    suf[5] = us[5]
    for k in range(4, 0, -1):
        suf[k] = suf[k + 1] * us[k]
    exts = []
    for k in range(6):
        if k == 0:
            exts.append(suf[1])
        elif k == 5:
            exts.append(pre[4])
        else:
            exts.append(pre[k - 1] * suf[k + 1])
    ext = jnp.stack(exts, axis=1).reshape(_ABLK, _BP)
    y = jnp.clip(ext, -1.0 + 1e-7, 1.0 - 1e-7)
    o_ref[...] = w * jnp.log((1.0 + y) / (1.0 - y))


def _boxplus_tc(g, w2d, prev=None):
    blk = pl.BlockSpec((_ABLK, _BP), lambda i: (i, 0))
    wblk = pl.BlockSpec((_ABLK, 1), lambda i: (i, 0))
    if prev is None:
        return pl.pallas_call(
            _boxplus_tc_kernel,
            out_shape=jax.ShapeDtypeStruct((_EP, _BP), jnp.float32),
            grid=(_EP // _ABLK,),
            in_specs=[blk, wblk],
            out_specs=blk,
        )(g, w2d)
    return pl.pallas_call(
        _boxplus_sub_tc_kernel,
        out_shape=jax.ShapeDtypeStruct((_EP, _BP), jnp.float32),
        grid=(_EP // _ABLK,),
        in_specs=[blk, blk, wblk],
        out_specs=blk,
    )(g, prev, w2d)


def _vnupd_tc_kernel(h_ref, l_ref, marg_ref, lp_ref):
    i = pl.program_id(0)
    vrows = _VBLK // 3
    h = h_ref[...]
    l = l_ref[...]
    h3 = h.reshape(vrows, 3, _BP)
    marg = l + jnp.sum(h3, axis=1)
    marg_ref[...] = marg
    rows = lax.broadcasted_iota(jnp.int32, (vrows, _BP), 0) + i * vrows
    cols = lax.broadcasted_iota(jnp.int32, (vrows, _BP), 1)
    mask = (rows < _N_VN) & (cols < _B)
    sp = jnp.where(mask, jax.nn.softplus(marg), 0.0)

    @pl.when(i == 0)
    def _():
        lp_ref[...] = jnp.zeros_like(lp_ref)

    lp_ref[...] += jnp.sum(sp, axis=0, keepdims=True)


def _vnupd_tc(h, llr_t):
    vrows = _VBLK // 3
    return pl.pallas_call(
        _vnupd_tc_kernel,
        out_shape=(
            jax.ShapeDtypeStruct((_VNP, _BP), jnp.float32),
            jax.ShapeDtypeStruct((1, _BP), jnp.float32),
        ),
        grid=(_EP // _VBLK,),
        in_specs=[
            pl.BlockSpec((_VBLK, _BP), lambda i: (i, 0)),
            pl.BlockSpec((vrows, _BP), lambda i: (i, 0)),
        ],
        out_specs=(
            pl.BlockSpec((vrows, _BP), lambda i: (i, 0)),
            pl.BlockSpec((1, _BP), lambda i: (0, 0)),
        ),
    )(h, llr_t)


def kernel(llr, weights, vn_idx, cn_idx):
    llr = llr.astype(jnp.float32)
    b = llr.shape[0]

    perm = jnp.argsort(cn_idx).astype(jnp.int32)
    inv = jnp.zeros((_E,), jnp.int32).at[perm].set(jnp.arange(_E, dtype=jnp.int32))
    zpad = jnp.zeros((_EP - _E,), jnp.int32)
    perm3_pad = jnp.concatenate([vn_idx[perm].astype(jnp.int32), zpad])
    inv_pad = jnp.concatenate([inv, zpad])
    w2d = jnp.concatenate([weights[perm].astype(jnp.float32),
                           jnp.ones((_EP - _E,), jnp.float32)])[:, None]

    llr_t = jnp.pad(llr.T, ((0, _VNP - _N_VN), (0, _BP - b)))

    gather_marg = _make_gather(_VNP)
    gather_back = _make_gather(_EP)

    loss_parts = []
    marg = None
    msg_cn = None
    for it in range(_NUM_ITER):
        if it == 0:
            g = gather_marg(llr_t, perm3_pad)
            msg_cn = _boxplus_tc(g, w2d)
        else:
            g = gather_marg(marg, perm3_pad)
            msg_cn = _boxplus_tc(g, w2d, prev=msg_cn)
        h = gather_back(msg_cn, inv_pad)
        marg, lp = _vnupd_tc(h, llr_t)
        loss_parts.append(lp)

    loss = jnp.sum(jnp.stack(loss_parts)) / jnp.float32(_NUM_ITER * b * _N_VN)
    c_hat = marg[:_N_VN, :b].T
    c = jnp.zeros((b, _N_VN), dtype=jnp.float32)
    return (c, c_hat, llr, loss)
